# baseline (speedup 1.0000x reference)
import functools

import jax
import jax.numpy as jnp
from jax import lax
from jax.experimental import pallas as pl
from jax.experimental.pallas import tpu as pltpu
from jax.experimental.pallas import tpu_sc as plsc

_NC = 2
_NS = 16
_NW = _NC * _NS


def _sc_aggregate(x, edges_il):
    N, D = x.shape
    NCHT, _, CH = edges_il.shape
    NCH = NCHT // _NW
    EW = NCH * CH
    RPT = (N // _NS) // 8 * 8
    RPT_LAST = N - (_NS - 1) * RPT
    mesh = plsc.VectorSubcoreMesh(core_axis_name="c", subcore_axis_name="s")

    @functools.partial(
        pl.kernel,
        out_type=jax.ShapeDtypeStruct((_NC, N, D), jnp.float32),
        mesh=mesh,
        scratch_types=(
            [pltpu.VMEM((2, CH), jnp.int32)] * 4
            + [pltpu.VMEM((CH, D), jnp.float32)] * 4
            + [pltpu.VMEM_SHARED((N, D), jnp.float32)]
            + [pltpu.SemaphoreType.DMA] * 12
        ),
    )
    def agg_kernel(x_hbm, edges_hbm, out_hbm,
                   i0, i1, i2, i3, r0, r1, r2, r3, acc_sh, *sems):
        cid = lax.axis_index("c")
        sid = lax.axis_index("s")
        wid = cid * _NS + sid
        idx = (i0, i1, i2, i3)
        sidx = tuple(b.at[0] for b in idx)
        didx = tuple(b.at[1] for b in idx)
        rows = (r0, r1, r2, r3)
        isem = sems[0:4]
        gsem = sems[4:8]
        ssem = sems[8:12]
        rows_v = r0
        cbase = wid * NCH

        def load_idx(c, k):
            pltpu.async_copy(edges_hbm.at[cbase + c], idx[k], isem[k])

        def wait_idx(k):
            pltpu.make_async_copy(edges_hbm.at[0], idx[k], isem[k]).wait()

        def wait_rows(k):
            pltpu.make_async_copy(x_hbm.at[pl.ds(0, CH)], rows[k],
                                  gsem[k]).wait()

        for k in range(4):
            load_idx(k, k)

        zeros16 = jnp.zeros((16,), jnp.float32)

        def zrow(i, carry):
            for j in range(D // 16):
                rows_v[i, pl.ds(j * 16, 16)] = zeros16
            return carry

        lax.fori_loop(0, CH, zrow, 0)

        @pl.when(sid < _NS - 1)
        def _zero_main():
            for k in range(RPT // CH):
                pltpu.sync_copy(rows_v,
                                acc_sh.at[pl.ds(sid * RPT + k * CH, CH)])
            rem = RPT % CH
            if rem:
                pltpu.sync_copy(
                    rows_v.at[pl.ds(0, rem)],
                    acc_sh.at[pl.ds(sid * RPT + (RPT // CH) * CH, rem)])

        @pl.when(sid == _NS - 1)
        def _zero_last():
            base = (_NS - 1) * RPT
            for k in range(RPT_LAST // CH):
                pltpu.sync_copy(rows_v, acc_sh.at[pl.ds(base + k * CH, CH)])
            rem = RPT_LAST % CH
            if rem:
                pltpu.sync_copy(
                    rows_v.at[pl.ds(0, rem)],
                    acc_sh.at[pl.ds(base + (RPT_LAST // CH) * CH, rem)])

        plsc.subcore_barrier()

        for k in range(4):
            wait_idx(k)
            pltpu.async_copy(x_hbm.at[sidx[k]], rows[k], gsem[k])

        def group(i, carry):
            c = 4 * i
            scat = []
            for k in range(4):
                wait_rows(k)
                scat.append(pltpu.async_copy(
                    rows[k], acc_sh.at[didx[k]], ssem[k], add=True))
            for k in range(4):
                scat[k].wait()

                @pl.when(c + k + 4 < NCH)
                def _(k=k):
                    load_idx(c + k + 4, k)

            for k in range(4):
                @pl.when(c + k + 4 < NCH)
                def _(k=k):
                    wait_idx(k)
                    pltpu.async_copy(x_hbm.at[sidx[k]], rows[k], gsem[k])

            return carry

        lax.fori_loop(0, NCH // 4, group, 0)
        wait_rows(0)
        pltpu.sync_copy(rows[0], acc_sh.at[didx[0]], add=True)
        plsc.subcore_barrier()

        @pl.when(sid < _NS - 1)
        def _out_main():
            pltpu.sync_copy(acc_sh.at[pl.ds(sid * RPT, RPT)],
                            out_hbm.at[cid, pl.ds(sid * RPT, RPT)])

        @pl.when(sid == _NS - 1)
        def _out_last():
            base = (_NS - 1) * RPT
            pltpu.sync_copy(acc_sh.at[pl.ds(base, RPT_LAST)],
                            out_hbm.at[cid, pl.ds(base, RPT_LAST)])

    return agg_kernel(x, edges_il)


def _tc_mlp(x, agg, W1, b1, W2, b2):
    N, D = x.shape
    H = W1.shape[1]
    BR = 2000

    def body(x_ref, a_ref, w1_ref, b1_ref, w2_ref, b2_ref, o_ref):
        s = x_ref[...] + a_ref[0] + a_ref[1]
        h = jnp.dot(s, w1_ref[...], preferred_element_type=jnp.float32)
        h = jnp.maximum(h + b1_ref[...], 0.0)
        o_ref[...] = (jnp.dot(h, w2_ref[...],
                              preferred_element_type=jnp.float32) + b2_ref[...])

    return pl.pallas_call(
        body,
        grid=(N // BR,),
        in_specs=[
            pl.BlockSpec((BR, D), lambda i: (i, 0)),
            pl.BlockSpec((_NC, BR, D), lambda i: (0, i, 0)),
            pl.BlockSpec((D, H), lambda i: (0, 0)),
            pl.BlockSpec((1, H), lambda i: (0, 0)),
            pl.BlockSpec((H, H), lambda i: (0, 0)),
            pl.BlockSpec((1, H), lambda i: (0, 0)),
        ],
        out_specs=pl.BlockSpec((BR, H), lambda i: (i, 0)),
        out_shape=jax.ShapeDtypeStruct((N, H), jnp.float32),
    )(x, agg, W1, b1, W2, b2)


def _sc_gather(x, uidx, vidx):
    N, D = x.shape
    B = uidx.shape[0]
    BW = B // _NW
    mesh = plsc.VectorSubcoreMesh(core_axis_name="c", subcore_axis_name="s")

    @functools.partial(
        pl.kernel,
        out_type=(jax.ShapeDtypeStruct((B, D), jnp.float32),
                  jax.ShapeDtypeStruct((B, D), jnp.float32)),
        mesh=mesh,
        scratch_types=[
            pltpu.VMEM((BW,), jnp.int32),
            pltpu.VMEM((BW, D), jnp.float32),
            pltpu.SemaphoreType.DMA,
        ],
    )
    def gather_kernel(x_hbm, u_hbm, v_hbm, ou_hbm, ov_hbm, idx_v, rows_v, sem):
        cid = lax.axis_index("c")
        sid = lax.axis_index("s")
        base = (cid * _NS + sid) * BW
        pltpu.sync_copy(u_hbm.at[pl.ds(base, BW)], idx_v)
        pltpu.async_copy(x_hbm.at[idx_v], rows_v, sem).wait()
        pltpu.sync_copy(rows_v, ou_hbm.at[pl.ds(base, BW)])
        pltpu.sync_copy(v_hbm.at[pl.ds(base, BW)], idx_v)
        pltpu.async_copy(x_hbm.at[idx_v], rows_v, sem).wait()
        pltpu.sync_copy(rows_v, ov_hbm.at[pl.ds(base, BW)])

    return gather_kernel(x, uidx, vidx)


def _tc_score(xu, xv, fu, fv, tb, ep_W1, ep_b1, ep_W2, ep_b2,
              te_W1, te_b1, te_W2, te_b2):
    Cn, H = xu.shape

    def body(xu_ref, xv_ref, fu_ref, fv_ref, tb_ref, w1_ref, b1_ref, w2_ref,
             b2_ref, tw1_ref, tb1_ref, tw2_ref, tb2_ref, o_ref):
        w1 = w1_ref[...]
        s = xu_ref[...] + xv_ref[...]
        d = jnp.abs(xu_ref[...] - xv_ref[...])
        ffs = fu_ref[...] + fv_ref[...]
        ffd = jnp.abs(fu_ref[...] - fv_ref[...])
        te = jnp.maximum(tb_ref[...] * tw1_ref[...] + tb1_ref[...], 0.0)
        temb = (jnp.dot(te, tw2_ref[...], preferred_element_type=jnp.float32)
                + tb2_ref[...])
        cvec = (jnp.dot(ffs, w1[0:H, :], preferred_element_type=jnp.float32)
                + jnp.dot(ffd, w1[H:2 * H, :],
                          preferred_element_type=jnp.float32)
                + jnp.dot(temb, w1[4 * H:5 * H, :],
                          preferred_element_type=jnp.float32)
                + b1_ref[...])
        pre = (jnp.dot(s, w1[2 * H:3 * H, :],
                       preferred_element_type=jnp.float32)
               + jnp.dot(d, w1[3 * H:4 * H, :],
                         preferred_element_type=jnp.float32)
               + cvec)
        h = jnp.maximum(pre, 0.0)
        o_ref[...] = (jnp.dot(h, w2_ref[...],
                              preferred_element_type=jnp.float32) + b2_ref[...])

    return pl.pallas_call(
        body,
        out_shape=jax.ShapeDtypeStruct((Cn, 1), jnp.float32),
    )(xu, xv, fu, fv, tb, ep_W1, ep_b1, ep_W2, ep_b2,
      te_W1, te_b1, te_W2, te_b2)


def kernel(x, edge_index, first_edge, candidate_edges, t,
           gin0_W1, gin0_b1, gin0_W2, gin0_b2,
           gin1_W1, gin1_b1, gin1_W2, gin1_b2,
           gin2_W1, gin2_b1, gin2_W2, gin2_b2,
           ep_W1, ep_b1, ep_W2, ep_b2,
           te_W1, te_b1, te_W2, te_b2):
    N, D = x.shape
    H = gin0_W1.shape[1]
    E = edge_index.shape[1]
    CH = 80
    NCHT = E // CH
    edges_il = jnp.stack([edge_index[0].reshape(NCHT, CH),
                          edge_index[1].reshape(NCHT, CH)], axis=1)

    layers = ((gin0_W1, gin0_b1, gin0_W2, gin0_b2),
              (gin1_W1, gin1_b1, gin1_W2, gin1_b2),
              (gin2_W1, gin2_b1, gin2_W2, gin2_b2))
    for W1, b1, W2, b2 in layers:
        agg = _sc_aggregate(x, edges_il)
        x = _tc_mlp(x, agg, W1, b1.reshape(1, H), W2, b2.reshape(1, H))

    Cn = candidate_edges.shape[0]
    B = ((Cn + 1 + 8 * _NW - 1) // (8 * _NW)) * (8 * _NW)
    pad = jnp.arange(B - Cn - 1, dtype=jnp.int32) % N
    uidx = jnp.concatenate([candidate_edges[:, 0].astype(jnp.int32),
                            first_edge[0:1].astype(jnp.int32), pad])
    vidx = jnp.concatenate([candidate_edges[:, 1].astype(jnp.int32),
                            first_edge[1:2].astype(jnp.int32), pad])
    xu_all, xv_all = _sc_gather(x, uidx, vidx)
    xu = xu_all[:Cn]
    xv = xv_all[:Cn]
    fu = xu_all[Cn:Cn + 1]
    fv = xv_all[Cn:Cn + 1]

    tb = jnp.full((1, H), t, dtype=jnp.float32)
    scores = _tc_score(xu, xv, fu, fv, tb,
                       ep_W1, ep_b1.reshape(1, H), ep_W2,
                       ep_b2.reshape(1, 1),
                       te_W1, te_b1.reshape(1, H), te_W2,
                       te_b2.reshape(1, H))
    return scores.reshape(-1)
